# baseline (speedup 1.0000x reference)
import functools

import jax
import jax.numpy as jnp
from jax import lax
from jax.experimental import pallas as pl
from jax.experimental.pallas import tpu as pltpu
from jax.experimental.pallas import tpu_sc as plsc

N_FIELDS = 26
VOCAB = 100000
EMB = 16
EPS = 1e-5

NW = 32
CHUNK = 128
NSTRIPE = 4


def _gather_kernel(B):
    epw = B // NW
    ncc = epw // CHUNK
    mesh = plsc.VectorSubcoreMesh(core_axis_name="c", subcore_axis_name="s")

    @functools.partial(
        pl.kernel,
        out_type=[jax.ShapeDtypeStruct((B, 128), jnp.float32)
                  for _ in range(NSTRIPE)],
        mesh=mesh,
        compiler_params=pltpu.CompilerParams(use_tc_tiling_on_sc=False),
        scratch_types=[
            pltpu.VMEM((N_FIELDS * ncc, CHUNK), jnp.int32),
            pltpu.VMEM((2, epw, EMB), jnp.float32),
            pltpu.SemaphoreType.DMA,
            pltpu.SemaphoreType.DMA,
            pltpu.SemaphoreType.DMA,
        ],
    )
    def gk(*refs):
        tabs = refs[:N_FIELDS]
        idx_hbm = refs[N_FIELDS]
        outs = refs[N_FIELDS + 1:N_FIELDS + 1 + NSTRIPE]
        idx_v, rows_v, gsem, wsem0, wsem1 = refs[N_FIELDS + 1 + NSTRIPE:]
        wid = lax.axis_index("s") * 2 + lax.axis_index("c")
        b0 = wid * epw
        pltpu.sync_copy(idx_hbm.at[wid], idx_v)

        wsems = (wsem0, wsem1)
        writes = [None, None]
        for f in range(N_FIELDS):
            buf = f % 2
            gathers = [
                pltpu.async_copy(
                    tabs[f].at[idx_v.at[f * ncc + cc]],
                    rows_v.at[buf, pl.ds(cc * CHUNK, CHUNK)],
                    gsem,
                )
                for cc in range(ncc)
            ]
            if writes[buf] is not None:
                writes[buf].wait()
            for g in gathers:
                g.wait()
            writes[buf] = pltpu.async_copy(
                rows_v.at[buf],
                outs[f // 8].at[pl.ds(b0, epw), pl.ds(EMB * (f % 8), EMB)],
                wsems[buf],
            )
        for w in writes:
            if w is not None:
                w.wait()

    return gk


def _mlp_body(xn_ref, e0_ref, e1_ref, e2_ref, e3_ref, w0n_ref, w0a_ref,
              w0b_ref, w0c_ref, w0d_ref, b0_ref, w1_ref, b1_ref,
              w2_ref, b2_ref, w3_ref, b3_ref, out_ref):
    h = jnp.dot(e0_ref[...], w0a_ref[...], preferred_element_type=jnp.float32)
    h = h + jnp.dot(e1_ref[...], w0b_ref[...], preferred_element_type=jnp.float32)
    h = h + jnp.dot(e2_ref[...], w0c_ref[...], preferred_element_type=jnp.float32)
    h = h + jnp.dot(e3_ref[...][:, :32], w0d_ref[...],
                    preferred_element_type=jnp.float32)
    h = h + jnp.dot(xn_ref[...], w0n_ref[...], preferred_element_type=jnp.float32)
    h = jnp.maximum(h + b0_ref[...], 0.0)
    h = jnp.maximum(jnp.dot(h, w1_ref[...], preferred_element_type=jnp.float32)
                    + b1_ref[...], 0.0)
    h = jnp.maximum(jnp.dot(h, w2_ref[...], preferred_element_type=jnp.float32)
                    + b2_ref[...], 0.0)
    out_ref[...] = jnp.dot(h, w3_ref[...], preferred_element_type=jnp.float32) + b3_ref[...]


def kernel(x_num, x_cat, tables, W0, b0, g0, be0, W1, b1, g1, be1,
           W2, b2, g2, be2, W3, b3):
    B = x_num.shape[0]
    epw = B // NW
    ncc = epw // CHUNK

    idx = (x_cat.astype(jnp.int32)
           .reshape(NW, epw, N_FIELDS)
           .transpose(0, 2, 1)
           .reshape(NW, N_FIELDS * ncc, CHUNK))

    inv = 1.0 / jnp.sqrt(1.0 + EPS)
    s0, s1, s2 = g0 * inv, g1 * inv, g2 * inv
    W0f = W0 * s0[None, :]
    b0f = (b0 * s0 + be0)[None, :]
    W1f = W1 * s1[None, :]
    b1f = (b1 * s1 + be1)[None, :]
    W2f = W2 * s2[None, :]
    b2f = (b2 * s2 + be2)[None, :]
    W0n = jnp.pad(W0f[:13], ((0, 3), (0, 0)))
    W0e = W0f[13:]
    W0a, W0b, W0c, W0d = (W0e[:128], W0e[128:256], W0e[256:384], W0e[384:])
    w3p = jnp.pad(W3, ((0, 0), (0, 127)))
    b3p = jnp.pad(b3.reshape(1, 1), ((0, 0), (0, 127)))
    xn = jnp.pad(x_num, ((0, 0), (0, 3)))

    tabs = [tables[f] for f in range(N_FIELDS)]
    e0, e1, e2, e3 = _gather_kernel(B)(*tabs, idx)

    BM = 2048
    nb = B // BM
    blk = lambda s: pl.BlockSpec(s, lambda i: (i, 0))
    full = lambda s: pl.BlockSpec(s, lambda i: (0, 0))
    out2 = pl.pallas_call(
        _mlp_body,
        grid=(nb,),
        in_specs=[
            blk((BM, 16)),
            blk((BM, 128)), blk((BM, 128)), blk((BM, 128)), blk((BM, 128)),
            full((16, 128)), full((128, 128)), full((128, 128)),
            full((128, 128)), full((32, 128)), full((1, 128)),
            full((128, 64)), full((1, 64)),
            full((64, 32)), full((1, 32)),
            full((32, 128)), full((1, 128)),
        ],
        out_specs=pl.BlockSpec((BM, 128), lambda i: (i, 0)),
        out_shape=jax.ShapeDtypeStruct((B, 128), jnp.float32),
    )(xn, e0, e1, e2, e3, W0n, W0a, W0b, W0c, W0d, b0f,
      W1f, b1f, W2f, b2f, w3p, b3p)

    return out2[:, 0]


# speedup vs baseline: 9.3361x; 9.3361x over previous
import functools

import jax
import jax.numpy as jnp
from jax import lax
from jax.experimental import pallas as pl
from jax.experimental.pallas import tpu as pltpu
from jax.experimental.pallas import tpu_sc as plsc

N_FIELDS = 26
VOCAB = 100000
EMB = 16
EPS = 1e-5

NW = 32
NROWS = N_FIELDS * EMB
RPW = NROWS // NW


def _gather_kernel(B):
    mesh = plsc.VectorSubcoreMesh(core_axis_name="c", subcore_axis_name="s")

    @functools.partial(
        pl.kernel,
        out_type=jax.ShapeDtypeStruct((NROWS, B), jnp.float32),
        mesh=mesh,
        compiler_params=pltpu.CompilerParams(use_tc_tiling_on_sc=True,
                                             needs_layout_passes=False),
        scratch_types=[
            pltpu.VMEM((VOCAB,), jnp.float32),
            pltpu.VMEM((B // NW * NW,), jnp.float32),
            pltpu.SemaphoreType.DMA,
            pltpu.SemaphoreType.DMA,
        ],
    )
    def gk(tt_hbm, idxf_hbm, out_hbm, row_v, io_v, rsem, isem):
        wid = lax.axis_index("s") * 2 + lax.axis_index("c")
        row0 = wid * RPW
        for k in range(RPW):
            row = row0 + k
            f = lax.shift_right_logical(row, 4)
            cr = pltpu.async_copy(tt_hbm.at[row], row_v, rsem)
            ci = pltpu.async_copy(idxf_hbm.at[f], io_v, isem)
            cr.wait()
            ci.wait()

            def blk(kk, carry):
                for j in range(16):
                    off = kk * 256 + j * 16
                    iv = plsc.bitcast(io_v[pl.ds(off, 16)], jnp.int32)
                    io_v[pl.ds(off, 16)] = plsc.load_gather(row_v, [iv])
                return carry

            lax.fori_loop(0, B // 256, blk, 0)
            pltpu.sync_copy(io_v, out_hbm.at[row])

    return gk


def _mlp_body(xn_ref, emb_ref, w0n_ref, w0e_ref, b0_ref, w1_ref, b1_ref,
              w2_ref, b2_ref, w3_ref, b3_ref, out_ref):
    dg = lambda a, b, ca, cb: lax.dot_general(
        a, b, (((ca,), (cb,)), ((), ())),
        preferred_element_type=jnp.float32)
    h = dg(w0e_ref[...], emb_ref[...], 0, 0)
    h = h + dg(w0n_ref[...], xn_ref[...], 0, 1)
    h = jnp.maximum(h + b0_ref[...], 0.0)
    h = jnp.maximum(dg(w1_ref[...], h, 0, 0) + b1_ref[...], 0.0)
    h = jnp.maximum(dg(w2_ref[...], h, 0, 0) + b2_ref[...], 0.0)
    out_ref[...] = dg(w3_ref[...], h, 0, 0) + b3_ref[...]


def kernel(x_num, x_cat, tables, W0, b0, g0, be0, W1, b1, g1, be1,
           W2, b2, g2, be2, W3, b3):
    B = x_num.shape[0]

    tt = tables.transpose(0, 2, 1).reshape(NROWS, VOCAB)
    idxf = lax.bitcast_convert_type(x_cat.astype(jnp.int32).T, jnp.float32)

    inv = 1.0 / jnp.sqrt(1.0 + EPS)
    s0, s1, s2 = g0 * inv, g1 * inv, g2 * inv
    W0f = W0 * s0[None, :]
    b0T = (b0 * s0 + be0)[:, None]
    W1f = W1 * s1[None, :]
    b1T = (b1 * s1 + be1)[:, None]
    W2f = W2 * s2[None, :]
    b2T = (b2 * s2 + be2)[:, None]
    W0n = jnp.pad(W0f[:13], ((0, 3), (0, 0)))
    W0e = W0f[13:]
    w3p = jnp.pad(W3, ((0, 0), (0, 7)))
    b3T = jnp.pad(b3.reshape(1, 1), ((0, 7), (0, 0)))
    xn = jnp.pad(x_num, ((0, 0), (0, 3)))

    embT = _gather_kernel(B)(tt, idxf)

    BM = 2048
    nb = B // BM
    full = lambda s: pl.BlockSpec(s, lambda i: (0, 0))
    out2 = pl.pallas_call(
        _mlp_body,
        grid=(nb,),
        in_specs=[
            pl.BlockSpec((BM, 16), lambda i: (i, 0)),
            pl.BlockSpec((NROWS, BM), lambda i: (0, i)),
            full((16, 128)), full((NROWS, 128)), full((128, 1)),
            full((128, 64)), full((64, 1)),
            full((64, 32)), full((32, 1)),
            full((32, 8)), full((8, 1)),
        ],
        out_specs=pl.BlockSpec((8, BM), lambda i: (0, i)),
        out_shape=jax.ShapeDtypeStruct((8, B), jnp.float32),
    )(xn, embT, W0n, W0e, b0T, W1f, b1T, W2f, b2T, w3p, b3T)

    return out2[0, :]


# speedup vs baseline: 9.5737x; 1.0254x over previous
import functools

import jax
import jax.numpy as jnp
from jax import lax
from jax.experimental import pallas as pl
from jax.experimental.pallas import tpu as pltpu
from jax.experimental.pallas import tpu_sc as plsc

N_FIELDS = 26
VOCAB = 100000
EMB = 16
EPS = 1e-5

NW = 32
NROWS = N_FIELDS * EMB
RPW = NROWS // NW


def _gather_kernel(B):
    mesh = plsc.VectorSubcoreMesh(core_axis_name="c", subcore_axis_name="s")

    @functools.partial(
        pl.kernel,
        out_type=jax.ShapeDtypeStruct((NROWS, B), jnp.float32),
        mesh=mesh,
        compiler_params=pltpu.CompilerParams(use_tc_tiling_on_sc=True,
                                             needs_layout_passes=False),
        scratch_types=[
            pltpu.VMEM((VOCAB,), jnp.float32),
            pltpu.VMEM((B // NW * NW,), jnp.float32),
            pltpu.SemaphoreType.DMA,
            pltpu.SemaphoreType.DMA,
            pltpu.SemaphoreType.DMA,
        ],
    )
    def gk(tt_hbm, idxf_hbm, out_hbm, row_v, io_v, rsem, isem, wsem):
        wid = lax.axis_index("s") * 2 + lax.axis_index("c")
        row0 = wid * RPW

        def row_dma(k):
            return pltpu.async_copy(tt_hbm.at[row0 + k], row_v, rsem)

        def idx_dma(k):
            f = lax.shift_right_logical(row0 + k, 4)
            return pltpu.async_copy(idxf_hbm.at[f], io_v, isem)

        cr = row_dma(0)
        ci = idx_dma(0)
        for k in range(RPW):
            cr.wait()
            ci.wait()

            def blk(kk, carry):
                for j in range(16):
                    off = kk * 256 + j * 16
                    iv = plsc.bitcast(io_v[pl.ds(off, 16)], jnp.int32)
                    io_v[pl.ds(off, 16)] = plsc.load_gather(row_v, [iv])
                return carry

            lax.fori_loop(0, B // 256, blk, 0)
            if k + 1 < RPW:
                cr = row_dma(k + 1)
            wb = pltpu.async_copy(io_v, out_hbm.at[row0 + k], wsem)
            wb.wait()
            if k + 1 < RPW:
                ci = idx_dma(k + 1)

    return gk


def _mlp_body(xn_ref, emb_ref, w0n_ref, w0e_ref, b0_ref, w1_ref, b1_ref,
              w2_ref, b2_ref, w3_ref, b3_ref, out_ref):
    dg = lambda a, b, ca, cb: lax.dot_general(
        a, b, (((ca,), (cb,)), ((), ())),
        preferred_element_type=jnp.float32)
    h = dg(w0e_ref[...], emb_ref[...], 0, 0)
    h = h + dg(w0n_ref[...], xn_ref[...], 0, 1)
    h = jnp.maximum(h + b0_ref[...], 0.0)
    h = jnp.maximum(dg(w1_ref[...], h, 0, 0) + b1_ref[...], 0.0)
    h = jnp.maximum(dg(w2_ref[...], h, 0, 0) + b2_ref[...], 0.0)
    out_ref[...] = dg(w3_ref[...], h, 0, 0) + b3_ref[...]


def kernel(x_num, x_cat, tables, W0, b0, g0, be0, W1, b1, g1, be1,
           W2, b2, g2, be2, W3, b3):
    B = x_num.shape[0]

    tt = tables.transpose(0, 2, 1).reshape(NROWS, VOCAB)
    idxf = lax.bitcast_convert_type(x_cat.astype(jnp.int32).T, jnp.float32)

    inv = 1.0 / jnp.sqrt(1.0 + EPS)
    s0, s1, s2 = g0 * inv, g1 * inv, g2 * inv
    W0f = W0 * s0[None, :]
    b0T = (b0 * s0 + be0)[:, None]
    W1f = W1 * s1[None, :]
    b1T = (b1 * s1 + be1)[:, None]
    W2f = W2 * s2[None, :]
    b2T = (b2 * s2 + be2)[:, None]
    W0n = W0f[:13]
    W0e = W0f[13:]
    w3p = jnp.pad(W3, ((0, 0), (0, 7)))
    b3T = jnp.pad(b3.reshape(1, 1), ((0, 7), (0, 0)))

    embT = _gather_kernel(B)(tt, idxf)

    BM = 2048
    nb = B // BM
    full = lambda s: pl.BlockSpec(s, lambda i: (0, 0))
    out2 = pl.pallas_call(
        _mlp_body,
        grid=(nb,),
        in_specs=[
            pl.BlockSpec((BM, 13), lambda i: (i, 0)),
            pl.BlockSpec((NROWS, BM), lambda i: (0, i)),
            full((13, 128)), full((NROWS, 128)), full((128, 1)),
            full((128, 64)), full((64, 1)),
            full((64, 32)), full((32, 1)),
            full((32, 8)), full((8, 1)),
        ],
        out_specs=pl.BlockSpec((8, BM), lambda i: (0, i)),
        out_shape=jax.ShapeDtypeStruct((8, B), jnp.float32),
    )(x_num, embT, W0n, W0e, b0T, W1f, b1T, W2f, b2T, w3p, b3T)

    return out2[0, :]
